# baseline (speedup 1.0000x reference)
import functools

import jax
import jax.numpy as jnp
from jax import lax
from jax.experimental import pallas as pl
from jax.experimental.pallas import tpu as pltpu
from jax.experimental.pallas import tpu_sc as plsc

_B, _N, _FD = 16, 1024, 128
_NKEY = 2 * _N
_NTOT = 3 * _N
_QB = 256
_F32 = jnp.float32
_BF16 = jnp.bfloat16


def _kp_body(fcn_ref, fca_ref, fh_ref, x_ref, wq_ref, bq_ref, wk_ref,
             bk_ref, ft_out, ftn_out, xn_out):
    wq = wq_ref[...]
    bq = bq_ref[0]
    dn = (((1,), (1,)), ((), ()))
    ft_out[0, 0:_N] = lax.dot_general(fcn_ref[0], wq, dn,
                                      preferred_element_type=_F32) + bq
    ft_out[0, _N:_NKEY] = lax.dot_general(fca_ref[0], wq, dn,
                                          preferred_element_type=_F32) + bq
    ft_out[0, _NKEY:_NTOT] = lax.dot_general(
        fh_ref[0], wk_ref[...], dn, preferred_element_type=_F32) + bk_ref[0]
    f = ft_out[0]
    ftn_out[0] = (f * lax.rsqrt(jnp.sum(f * f, axis=1, keepdims=True))
                  ).astype(_BF16)
    x = x_ref[0]
    xn_out[0] = (x * lax.rsqrt(jnp.sum(x * x, axis=1, keepdims=True))
                 ).astype(_BF16)


_kp_call = pl.pallas_call(
    _kp_body,
    grid=(_B,),
    in_specs=[
        pl.BlockSpec((1, _N, _FD), lambda b: (b, 0, 0)),
        pl.BlockSpec((1, _N, _FD), lambda b: (b, 0, 0)),
        pl.BlockSpec((1, _N, _FD), lambda b: (b, 0, 0)),
        pl.BlockSpec((1, _N, _FD), lambda b: (b, 0, 0)),
        pl.BlockSpec((_FD, _FD), lambda b: (0, 0)),
        pl.BlockSpec((1, _FD), lambda b: (0, 0)),
        pl.BlockSpec((_FD, _FD), lambda b: (0, 0)),
        pl.BlockSpec((1, _FD), lambda b: (0, 0)),
    ],
    out_specs=[
        pl.BlockSpec((1, _NTOT, _FD), lambda b: (b, 0, 0)),
        pl.BlockSpec((1, _NTOT, _FD), lambda b: (b, 0, 0)),
        pl.BlockSpec((1, _N, _FD), lambda b: (b, 0, 0)),
    ],
    out_shape=[
        jax.ShapeDtypeStruct((_B, _NTOT, _FD), _F32),
        jax.ShapeDtypeStruct((_B, _NTOT, _FD), _BF16),
        jax.ShapeDtypeStruct((_B, _N, _FD), _BF16),
    ],
)


def _ka_body(ftn_ref, sabn_ref, ns_out):
    qb = pl.program_id(1)
    kn = ftn_ref[0, 0:_NKEY, :]
    for h in range(2):
        base = _NKEY + qb * _QB + h * (_QB // 2)
        qn = ftn_ref[0, pl.ds(base, _QB // 2), :]
        s = lax.dot_general(qn, kn, (((1,), (1,)), ((), ())),
                            preferred_element_type=_F32)
        e = jnp.exp(s)
        den = jnp.sum(e, axis=1)
        num = lax.dot_general(e[:, _N:_NKEY], sabn_ref[0, 0],
                              (((1,), (0,)), ((), ())),
                              preferred_element_type=_F32)
        ns_out[0, 0, pl.ds(qb * _QB + h * (_QB // 2), _QB // 2)] = num / den


_ka_call = pl.pallas_call(
    _ka_body,
    grid=(_B, _N // _QB),
    in_specs=[
        pl.BlockSpec((1, _NTOT, _FD), lambda b, q: (b, 0, 0)),
        pl.BlockSpec((1, 1, _N), lambda b, q: (b, 0, 0)),
    ],
    out_specs=pl.BlockSpec((1, 1, _N), lambda b, q: (b, 0, 0)),
    out_shape=jax.ShapeDtypeStruct((_B, 1, _N), _F32),
)


def _kb_body(x_ref, xn_ref, o_ref):
    qb = pl.program_id(1)
    for h in range(2):
        qn = xn_ref[0, pl.ds(qb * _QB + h * (_QB // 2), _QB // 2), :]
        s = lax.dot_general(qn, xn_ref[0], (((1,), (1,)), ((), ())),
                            preferred_element_type=_F32)
        e = jnp.exp(s)
        den = jnp.sum(e, axis=1, keepdims=True)
        o = lax.dot_general(e, x_ref[0], (((1,), (0,)), ((), ())),
                            preferred_element_type=_F32)
        o_ref[0, pl.ds(h * (_QB // 2), _QB // 2), :] = o / den


_kb_call = pl.pallas_call(
    _kb_body,
    grid=(_B, _N // _QB),
    in_specs=[
        pl.BlockSpec((1, _N, _FD), lambda b, q: (b, 0, 0)),
        pl.BlockSpec((1, _N, _FD), lambda b, q: (b, 0, 0)),
    ],
    out_specs=pl.BlockSpec((1, _QB, _FD), lambda b, q: (b, q, 0)),
    out_shape=jax.ShapeDtypeStruct((_B, _N, _FD), _F32),
)


def _kc_body(ns_hbm, sabn_hbm, ftf_hbm, topn_out, topa_out, gath_out,
             buf, nsb, gidx, rows, res, sem):
    wid = lax.axis_index("s") * 2 + lax.axis_index("c")

    @pl.when(wid < _B)
    def _work():
        b = wid
        iota16 = lax.iota(jnp.int32, 16)
        lane0 = iota16 == 0

        pltpu.sync_copy(sabn_hbm.at[b], buf.at[pl.ds(_N, _N)])
        pltpu.sync_copy(ns_hbm.at[b], buf.at[pl.ds(_NKEY, _N)])
        pltpu.sync_copy(ns_hbm.at[b], nsb)

        def _zero(j, c):
            buf[pl.ds(j * 16, 16)] = jnp.zeros((16,), _F32)
            return c
        lax.fori_loop(0, _N // 16, _zero, 0)

        bigi = jnp.int32(1 << 30)

        def _extract(ref, nvec, biggest):
            def body(j, c):
                bv, bi = c
                v = ref[pl.ds(j * 16, 16)]
                take = (v > bv) if biggest else (v < bv)
                gi = j * 16 + iota16
                return jnp.where(take, v, bv), jnp.where(take, gi, bi)
            init = jnp.full((16,), -3e38 if biggest else 3e38, _F32)
            bv, bi = lax.fori_loop(0, nvec, body,
                                   (init, jnp.zeros((16,), jnp.int32)))
            mval = bv[0]
            pick = jnp.maximum if biggest else jnp.minimum
            for i in range(1, 16):
                mval = pick(mval, bv[i])
            gi = bigi
            for i in range(16):
                gi = jnp.minimum(gi, jnp.where(bv[i] == mval, bi[i], bigi))
            return mval, gi

        def _put(ref, gi, val):
            g = (gi // 16) * 16
            lane = gi - g
            w = ref[pl.ds(g, 16)]
            ref[pl.ds(g, 16)] = jnp.where(iota16 == lane, val, w)

        a1, ai1 = _extract(nsb, _N // 16, True)
        _put(nsb, ai1, jnp.float32(-3e38))
        a2, _ = _extract(nsb, _N // 16, True)
        _put(nsb, ai1, a1)

        nv = []
        for _k in range(5):
            v, gi = _extract(nsb, _N // 16, False)
            nv.append(v)
            _put(nsb, gi, jnp.float32(3e38))

        gv = jnp.zeros((16,), jnp.int32)
        for _k in range(5):
            v, gi = _extract(buf, _NTOT // 16, True)
            gv = jnp.where(iota16 == _k, b * _NTOT + gi, gv)
            _put(buf, gi, jnp.float32(-3e38))

        rn = jnp.zeros((16,), _F32)
        for _k in range(5):
            rn = jnp.where(iota16 == _k, nv[_k], rn)
        res[...] = rn
        pltpu.sync_copy(res, topn_out.at[b])

        ra = jnp.where(lane0, a1, jnp.where(iota16 == 1, a2, 0.0))
        res[...] = ra
        pltpu.sync_copy(res, topa_out.at[b])

        gidx[...] = gv
        pltpu.async_copy(ftf_hbm.at[gidx], rows, sem).wait()
        pltpu.sync_copy(rows, gath_out.at[b])


@functools.lru_cache(maxsize=1)
def _kc_call():
    return pl.kernel(
        _kc_body,
        out_type=[
            jax.ShapeDtypeStruct((_B, 16), _F32),
            jax.ShapeDtypeStruct((_B, 16), _F32),
            jax.ShapeDtypeStruct((_B, 16, _FD), _F32),
        ],
        mesh=plsc.VectorSubcoreMesh(core_axis_name="c", subcore_axis_name="s"),
        scratch_types=[
            pltpu.VMEM((_NTOT,), _F32),
            pltpu.VMEM((_N,), _F32),
            pltpu.VMEM((16,), jnp.int32),
            pltpu.VMEM((16, _FD), _F32),
            pltpu.VMEM((16,), _F32),
            pltpu.SemaphoreType.DMA,
        ],
    )


def _kd_attn(x, nreal):
    bi = lax.broadcasted_iota(jnp.int32, (_B * 16, _B * 16), 0)
    bj = lax.broadcasted_iota(jnp.int32, (_B * 16, _B * 16), 1)
    mask = ((bi // 16) == (bj // 16)) & ((bj % 16) < nreal)
    n = x * lax.rsqrt(jnp.sum(x * x, axis=1, keepdims=True))
    s = lax.dot_general(n, n, (((1,), (1,)), ((), ())),
                        preferred_element_type=_F32)
    e = jnp.where(mask, jnp.exp(s), 0.0)
    den = jnp.sum(e, axis=1, keepdims=True)
    o = lax.dot_general(e, x, (((1,), (0,)), ((), ())),
                        preferred_element_type=_F32)
    return o / den


def _kd_body(xn_ref, xa_ref, on_ref, oa_ref):
    on_ref[...] = _kd_attn(xn_ref[...].reshape(_B * 16, _FD),
                           10).reshape(_B, 16, _FD)
    oa_ref[...] = _kd_attn(xa_ref[...].reshape(_B * 16, _FD),
                           5).reshape(_B, 16, _FD)


_kd_call = pl.pallas_call(
    _kd_body,
    grid=(1,),
    in_specs=[
        pl.BlockSpec((_B, 16, _FD), lambda s: (0, 0, 0)),
        pl.BlockSpec((_B, 16, _FD), lambda s: (0, 0, 0)),
    ],
    out_specs=[
        pl.BlockSpec((_B, 16, _FD), lambda s: (0, 0, 0)),
        pl.BlockSpec((_B, 16, _FD), lambda s: (0, 0, 0)),
    ],
    out_shape=[
        jax.ShapeDtypeStruct((_B, 16, _FD), _F32),
        jax.ShapeDtypeStruct((_B, 16, _FD), _F32),
    ],
)


def kernel(nor_feat_conf_nor, feat_conf_nor, score_conf_nor, feat_conf_abn,
           score_conf_abn, feat_hard, score_hard, Wq, bq, Wk, bk):
    ft, ftn, xn = _kp_call(feat_conf_nor, feat_conf_abn, feat_hard,
                           nor_feat_conf_nor, Wq, bq.reshape(1, _FD),
                           Wk, bk.reshape(1, _FD))
    ns = _ka_call(ftn, score_conf_abn.reshape(_B, 1, _N)).reshape(_B, _N)
    nnf = _kb_call(nor_feat_conf_nor, xn)
    topn, topa, gath = _kc_call()(ns, score_conf_abn,
                                  ft.reshape(_B * _NTOT, _FD))
    kdn, kda = _kd_call(ft, gath)
    return (ns, topn[:, :5], topa[:, :2], nnf, kdn[:, :10], kda[:, :5])


# speedup vs baseline: 1.0710x; 1.0710x over previous
import functools

import jax
import jax.numpy as jnp
from jax import lax
from jax.experimental import pallas as pl
from jax.experimental.pallas import tpu as pltpu
from jax.experimental.pallas import tpu_sc as plsc

_B, _N, _FD = 16, 1024, 128
_NKEY = 2 * _N
_NTOT = 3 * _N
_QB = 256
_F32 = jnp.float32
_BF16 = jnp.bfloat16


def _kp_body(fcn_ref, fca_ref, fh_ref, wq_ref, bq_ref, wk_ref,
             bk_ref, ftn_out, ft_s):
    wq = wq_ref[...]
    bq = bq_ref[0]
    dn = (((1,), (1,)), ((), ()))
    ft_s[0:_N] = lax.dot_general(fcn_ref[0], wq, dn,
                                 preferred_element_type=_F32) + bq
    ft_s[_N:_NKEY] = lax.dot_general(fca_ref[0], wq, dn,
                                     preferred_element_type=_F32) + bq
    ft_s[_NKEY:_NTOT] = lax.dot_general(
        fh_ref[0], wk_ref[...], dn, preferred_element_type=_F32) + bk_ref[0]
    f = ft_s[...]
    ftn_out[0] = (f * lax.rsqrt(jnp.sum(f * f, axis=1, keepdims=True))
                  ).astype(_BF16)


_kp_call = pl.pallas_call(
    _kp_body,
    grid=(_B,),
    in_specs=[
        pl.BlockSpec((1, _N, _FD), lambda b: (b, 0, 0)),
        pl.BlockSpec((1, _N, _FD), lambda b: (b, 0, 0)),
        pl.BlockSpec((1, _N, _FD), lambda b: (b, 0, 0)),
        pl.BlockSpec((_FD, _FD), lambda b: (0, 0)),
        pl.BlockSpec((1, _FD), lambda b: (0, 0)),
        pl.BlockSpec((_FD, _FD), lambda b: (0, 0)),
        pl.BlockSpec((1, _FD), lambda b: (0, 0)),
    ],
    out_specs=pl.BlockSpec((1, _NTOT, _FD), lambda b: (b, 0, 0)),
    out_shape=jax.ShapeDtypeStruct((_B, _NTOT, _FD), _BF16),
    scratch_shapes=[pltpu.VMEM((_NTOT, _FD), _F32)],
)


def _ka_body(ftn_ref, sabn_ref, ns_out):
    qb = pl.program_id(1)
    kn = ftn_ref[0, 0:_NKEY, :]
    for h in range(2):
        base = _NKEY + qb * _QB + h * (_QB // 2)
        qn = ftn_ref[0, pl.ds(base, _QB // 2), :]
        s = lax.dot_general(qn, kn, (((1,), (1,)), ((), ())),
                            preferred_element_type=_F32)
        e = jnp.exp(s)
        den = jnp.sum(e, axis=1)
        num = lax.dot_general(e[:, _N:_NKEY], sabn_ref[0, 0],
                              (((1,), (0,)), ((), ())),
                              preferred_element_type=_F32)
        ns_out[0, 0, pl.ds(qb * _QB + h * (_QB // 2), _QB // 2)] = num / den


_ka_call = pl.pallas_call(
    _ka_body,
    grid=(_B, _N // _QB),
    in_specs=[
        pl.BlockSpec((1, _NTOT, _FD), lambda b, q: (b, 0, 0)),
        pl.BlockSpec((1, 1, _N), lambda b, q: (b, 0, 0)),
    ],
    out_specs=pl.BlockSpec((1, 1, _N), lambda b, q: (b, 0, 0)),
    out_shape=jax.ShapeDtypeStruct((_B, 1, _N), _F32),
)


def _kb_body(x_ref, o_ref, xn_s):
    qb = pl.program_id(1)

    @pl.when(qb == 0)
    def _norm():
        x = x_ref[0]
        xn_s[...] = (x * lax.rsqrt(jnp.sum(x * x, axis=1, keepdims=True))
                     ).astype(_BF16)

    qn = xn_s[pl.ds(qb * _QB, _QB), :]
    s = lax.dot_general(qn, xn_s[...], (((1,), (1,)), ((), ())),
                        preferred_element_type=_F32)
    e = jnp.exp(s)
    den = jnp.sum(e, axis=1, keepdims=True)
    o = lax.dot_general(e, x_ref[0], (((1,), (0,)), ((), ())),
                        preferred_element_type=_F32)
    o_ref[0] = o / den


_kb_call = pl.pallas_call(
    _kb_body,
    grid=(_B, _N // _QB),
    in_specs=[pl.BlockSpec((1, _N, _FD), lambda b, q: (b, 0, 0))],
    out_specs=pl.BlockSpec((1, _QB, _FD), lambda b, q: (b, q, 0)),
    out_shape=jax.ShapeDtypeStruct((_B, _N, _FD), _F32),
    scratch_shapes=[pltpu.VMEM((_N, _FD), _BF16)],
)


def _kc_body(ns_hbm, sabn_hbm, fcn_hbm, fca_hbm, fh_hbm,
             topn_out, topa_out, gath_out, segf_out,
             buf, nsb, gidx, rows0, rows1, rows2, segx, res, sem):
    wid = lax.axis_index("s") * 2 + lax.axis_index("c")

    @pl.when(wid < _B)
    def _work():
        b = wid
        iota16 = lax.iota(jnp.int32, 16)
        lane0 = iota16 == 0

        pltpu.sync_copy(sabn_hbm.at[b], buf.at[pl.ds(_N, _N)])
        pltpu.sync_copy(ns_hbm.at[b], buf.at[pl.ds(_NKEY, _N)])
        pltpu.sync_copy(ns_hbm.at[b], nsb)

        def _zero(j, c):
            buf[pl.ds(j * 16, 16)] = jnp.zeros((16,), _F32)
            return c
        lax.fori_loop(0, _N // 16, _zero, 0)

        bigi = jnp.int32(1 << 30)

        def _extract(ref, nvec, biggest):
            def body(j, c):
                bv, bi = c
                v = ref[pl.ds(j * 16, 16)]
                take = (v > bv) if biggest else (v < bv)
                gi = j * 16 + iota16
                return jnp.where(take, v, bv), jnp.where(take, gi, bi)
            init = jnp.full((16,), -3e38 if biggest else 3e38, _F32)
            bv, bi = lax.fori_loop(0, nvec, body,
                                   (init, jnp.zeros((16,), jnp.int32)))
            mval = bv[0]
            pick = jnp.maximum if biggest else jnp.minimum
            for i in range(1, 16):
                mval = pick(mval, bv[i])
            gi = bigi
            for i in range(16):
                gi = jnp.minimum(gi, jnp.where(bv[i] == mval, bi[i], bigi))
            return mval, gi

        def _put(ref, gi, val):
            g = (gi // 16) * 16
            lane = gi - g
            w = ref[pl.ds(g, 16)]
            ref[pl.ds(g, 16)] = jnp.where(iota16 == lane, val, w)

        a1, ai1 = _extract(nsb, _N // 16, True)
        _put(nsb, ai1, jnp.float32(-3e38))
        a2, _ = _extract(nsb, _N // 16, True)
        _put(nsb, ai1, a1)

        nv = []
        for _k in range(5):
            v, gi = _extract(nsb, _N // 16, False)
            nv.append(v)
            _put(nsb, gi, jnp.float32(3e38))

        gv0 = jnp.zeros((16,), jnp.int32)
        gv1 = jnp.zeros((16,), jnp.int32)
        gv2 = jnp.zeros((16,), jnp.int32)
        sv = jnp.zeros((16,), _F32)
        for _k in range(5):
            v, gi = _extract(buf, _NTOT // 16, True)
            _put(buf, gi, jnp.float32(-3e38))
            seg = gi // _N
            addr = b * _N + (gi - seg * _N)
            lk = iota16 == _k
            gv0 = jnp.where(lk, jnp.where(seg == 0, addr, gv0[_k]), gv0)
            gv1 = jnp.where(lk, jnp.where(seg == 1, addr, gv1[_k]), gv1)
            gv2 = jnp.where(lk, jnp.where(seg == 2, addr, gv2[_k]), gv2)
            sv = jnp.where(lk, seg.astype(_F32), sv)

        rn = jnp.zeros((16,), _F32)
        for _k in range(5):
            rn = jnp.where(iota16 == _k, nv[_k], rn)
        res[...] = rn
        pltpu.sync_copy(res, topn_out.at[b])

        ra = jnp.where(lane0, a1, jnp.where(iota16 == 1, a2, 0.0))
        res[...] = ra
        pltpu.sync_copy(res, topa_out.at[b])

        gidx[...] = jnp.where(gv0 == 0, b * _N, gv0)
        pltpu.async_copy(fcn_hbm.at[gidx], rows0, sem).wait()
        gidx[...] = jnp.where(gv1 == 0, b * _N, gv1)
        pltpu.async_copy(fca_hbm.at[gidx], rows1, sem).wait()
        gidx[...] = jnp.where(gv2 == 0, b * _N, gv2)
        pltpu.async_copy(fh_hbm.at[gidx], rows2, sem).wait()
        for _r in range(16):
            sr = sv[_r]
            for _c in range(_FD // 16):
                c0 = rows0[_r, pl.ds(_c * 16, 16)]
                c1 = rows1[_r, pl.ds(_c * 16, 16)]
                c2 = rows2[_r, pl.ds(_c * 16, 16)]
                rows0[_r, pl.ds(_c * 16, 16)] = jnp.where(
                    sr == 1.0, c1, jnp.where(sr == 2.0, c2, c0))
        pltpu.sync_copy(rows0, gath_out.at[b])
        for _r in range(16):
            srow = jnp.broadcast_to(sv[_r], (16,))
            for _c in range(_FD // 16):
                segx[_r, pl.ds(_c * 16, 16)] = srow
        pltpu.sync_copy(segx, segf_out.at[b])


@functools.lru_cache(maxsize=1)
def _kc_call():
    return pl.kernel(
        _kc_body,
        out_type=[
            jax.ShapeDtypeStruct((_B, 16), _F32),
            jax.ShapeDtypeStruct((_B, 16), _F32),
            jax.ShapeDtypeStruct((_B, 16, _FD), _F32),
            jax.ShapeDtypeStruct((_B, 16, _FD), _F32),
        ],
        mesh=plsc.VectorSubcoreMesh(core_axis_name="c", subcore_axis_name="s"),
        scratch_types=[
            pltpu.VMEM((_NTOT,), _F32),
            pltpu.VMEM((_N,), _F32),
            pltpu.VMEM((16,), jnp.int32),
            pltpu.VMEM((16, _FD), _F32),
            pltpu.VMEM((16, _FD), _F32),
            pltpu.VMEM((16, _FD), _F32),
            pltpu.VMEM((16, _FD), _F32),
            pltpu.VMEM((16,), _F32),
            pltpu.SemaphoreType.DMA,
        ],
    )


def _kd_attn(x, nreal):
    bi = lax.broadcasted_iota(jnp.int32, (_B * 16, _B * 16), 0)
    bj = lax.broadcasted_iota(jnp.int32, (_B * 16, _B * 16), 1)
    mask = ((bi // 16) == (bj // 16)) & ((bj % 16) < nreal)
    n = x * lax.rsqrt(jnp.sum(x * x, axis=1, keepdims=True))
    s = lax.dot_general(n, n, (((1,), (1,)), ((), ())),
                        preferred_element_type=_F32)
    e = jnp.where(mask, jnp.exp(s), 0.0)
    den = jnp.sum(e, axis=1, keepdims=True)
    o = lax.dot_general(e, x, (((1,), (0,)), ((), ())),
                        preferred_element_type=_F32)
    return o / den


def _kd_body(fcn_ref, g_ref, seg_ref, wq_ref, bq_ref, wk_ref, bk_ref,
             on_ref, oa_ref):
    dn = (((1,), (1,)), ((), ()))
    wq = wq_ref[...]
    bq = bq_ref[0]
    xnor = lax.dot_general(fcn_ref[...].reshape(_B * 16, _FD), wq, dn,
                           preferred_element_type=_F32) + bq
    g = g_ref[...].reshape(_B * 16, _FD)
    pq = lax.dot_general(g, wq, dn, preferred_element_type=_F32) + bq
    pk = lax.dot_general(g, wk_ref[...], dn,
                         preferred_element_type=_F32) + bk_ref[0]
    segc = seg_ref[...].reshape(_B * 16, _FD)
    xabn = jnp.where(segc >= 1.5, pk, pq)
    on_ref[...] = _kd_attn(xnor, 10).reshape(_B, 16, _FD)
    oa_ref[...] = _kd_attn(xabn, 5).reshape(_B, 16, _FD)


_kd_call = pl.pallas_call(
    _kd_body,
    grid=(1,),
    in_specs=[
        pl.BlockSpec((_B, 16, _FD), lambda s: (0, 0, 0)),
        pl.BlockSpec((_B, 16, _FD), lambda s: (0, 0, 0)),
        pl.BlockSpec((_B, 16, _FD), lambda s: (0, 0, 0)),
        pl.BlockSpec((_FD, _FD), lambda s: (0, 0)),
        pl.BlockSpec((1, _FD), lambda s: (0, 0)),
        pl.BlockSpec((_FD, _FD), lambda s: (0, 0)),
        pl.BlockSpec((1, _FD), lambda s: (0, 0)),
    ],
    out_specs=[
        pl.BlockSpec((_B, 16, _FD), lambda s: (0, 0, 0)),
        pl.BlockSpec((_B, 16, _FD), lambda s: (0, 0, 0)),
    ],
    out_shape=[
        jax.ShapeDtypeStruct((_B, 16, _FD), _F32),
        jax.ShapeDtypeStruct((_B, 16, _FD), _F32),
    ],
)


def kernel(nor_feat_conf_nor, feat_conf_nor, score_conf_nor, feat_conf_abn,
           score_conf_abn, feat_hard, score_hard, Wq, bq, Wk, bk):
    ftn = _kp_call(feat_conf_nor, feat_conf_abn, feat_hard,
                   Wq, bq.reshape(1, _FD), Wk, bk.reshape(1, _FD))
    ns = _ka_call(ftn, score_conf_abn.reshape(_B, 1, _N)).reshape(_B, _N)
    nnf = _kb_call(nor_feat_conf_nor)
    topn, topa, gath, segf = _kc_call()(
        ns, score_conf_abn, feat_conf_nor.reshape(_B * _N, _FD),
        feat_conf_abn.reshape(_B * _N, _FD), feat_hard.reshape(_B * _N, _FD))
    kdn, kda = _kd_call(feat_conf_nor, gath, segf, Wq, bq.reshape(1, _FD),
                        Wk, bk.reshape(1, _FD))
    return (ns, topn[:, :5], topa[:, :2], nnf, kdn[:, :10], kda[:, :5])


# speedup vs baseline: 1.3573x; 1.2674x over previous
import functools

import jax
import jax.numpy as jnp
from jax import lax
from jax.experimental import pallas as pl
from jax.experimental.pallas import tpu as pltpu
from jax.experimental.pallas import tpu_sc as plsc

_B, _N, _FD = 16, 1024, 128
_NKEY = 2 * _N
_NTOT = 3 * _N
_QB = 512
_F32 = jnp.float32
_BF16 = jnp.bfloat16


def _kp_body(fcn_ref, fca_ref, fh_ref, wq_ref, bq_ref, wk_ref,
             bk_ref, ftn_out, ft_s):
    wq = wq_ref[...]
    bq = bq_ref[0]
    dn = (((1,), (1,)), ((), ()))
    ft_s[0:_N] = lax.dot_general(fcn_ref[0], wq, dn,
                                 preferred_element_type=_F32) + bq
    ft_s[_N:_NKEY] = lax.dot_general(fca_ref[0], wq, dn,
                                     preferred_element_type=_F32) + bq
    ft_s[_NKEY:_NTOT] = lax.dot_general(
        fh_ref[0], wk_ref[...], dn, preferred_element_type=_F32) + bk_ref[0]
    f = ft_s[...]
    ftn_out[0] = (f * lax.rsqrt(jnp.sum(f * f, axis=1, keepdims=True))
                  ).astype(_BF16)


_kp_call = pl.pallas_call(
    _kp_body,
    grid=(_B,),
    in_specs=[
        pl.BlockSpec((1, _N, _FD), lambda b: (b, 0, 0)),
        pl.BlockSpec((1, _N, _FD), lambda b: (b, 0, 0)),
        pl.BlockSpec((1, _N, _FD), lambda b: (b, 0, 0)),
        pl.BlockSpec((_FD, _FD), lambda b: (0, 0)),
        pl.BlockSpec((1, _FD), lambda b: (0, 0)),
        pl.BlockSpec((_FD, _FD), lambda b: (0, 0)),
        pl.BlockSpec((1, _FD), lambda b: (0, 0)),
    ],
    out_specs=pl.BlockSpec((1, _NTOT, _FD), lambda b: (b, 0, 0)),
    out_shape=jax.ShapeDtypeStruct((_B, _NTOT, _FD), _BF16),
    scratch_shapes=[pltpu.VMEM((_NTOT, _FD), _F32)],
)


def _ka_body(ftn_ref, sabn_ref, ns_out):
    qb = pl.program_id(1)
    kn = ftn_ref[0, 0:_NKEY, :]
    for h in range(2):
        base = _NKEY + qb * _QB + h * (_QB // 2)
        qn = ftn_ref[0, pl.ds(base, _QB // 2), :]
        s = lax.dot_general(qn, kn, (((1,), (1,)), ((), ())),
                            preferred_element_type=_F32)
        e = jnp.exp(s)
        den = jnp.sum(e, axis=1)
        num = lax.dot_general(e[:, _N:_NKEY], sabn_ref[0, 0],
                              (((1,), (0,)), ((), ())),
                              preferred_element_type=_F32)
        ns_out[0, 0, pl.ds(qb * _QB + h * (_QB // 2), _QB // 2)] = num / den


_ka_call = pl.pallas_call(
    _ka_body,
    grid=(_B, _N // _QB),
    in_specs=[
        pl.BlockSpec((1, _NTOT, _FD), lambda b, q: (b, 0, 0)),
        pl.BlockSpec((1, 1, _N), lambda b, q: (b, 0, 0)),
    ],
    out_specs=pl.BlockSpec((1, 1, _N), lambda b, q: (b, 0, 0)),
    out_shape=jax.ShapeDtypeStruct((_B, 1, _N), _F32),
)


def _kb_body(x_ref, o_ref, xn_s):
    qb = pl.program_id(1)

    @pl.when(qb == 0)
    def _norm():
        x = x_ref[0]
        xn_s[...] = (x * lax.rsqrt(jnp.sum(x * x, axis=1, keepdims=True))
                     ).astype(_BF16)

    qn = xn_s[pl.ds(qb * _QB, _QB), :]
    s = lax.dot_general(qn, xn_s[...], (((1,), (1,)), ((), ())),
                        preferred_element_type=_F32)
    e = jnp.exp(s)
    den = lax.dot_general(e, jnp.ones((_N,), _F32), (((1,), (0,)), ((), ())),
                          preferred_element_type=_F32)
    o = lax.dot_general(e, x_ref[0], (((1,), (0,)), ((), ())),
                        preferred_element_type=_F32)
    o_ref[0] = o / den[:, None]


_kb_call = pl.pallas_call(
    _kb_body,
    grid=(_B, _N // _QB),
    in_specs=[pl.BlockSpec((1, _N, _FD), lambda b, q: (b, 0, 0))],
    out_specs=pl.BlockSpec((1, _QB, _FD), lambda b, q: (b, q, 0)),
    out_shape=jax.ShapeDtypeStruct((_B, _N, _FD), _F32),
    scratch_shapes=[pltpu.VMEM((_N, _FD), _BF16)],
)


def _kc_body(ns_hbm, sabn_hbm, fcn_hbm, fca_hbm, fh_hbm,
             topn_out, topa_out, gath_out, segf_out,
             buf, nsb, gidx, rows0, rows1, rows2, segx, res, sem):
    wid = lax.axis_index("s") * 2 + lax.axis_index("c")

    @pl.when(wid < _B)
    def _work():
        b = wid
        iota16 = lax.iota(jnp.int32, 16)
        lane0 = iota16 == 0

        pltpu.sync_copy(sabn_hbm.at[b], buf.at[pl.ds(_N, _N)])
        pltpu.sync_copy(ns_hbm.at[b], buf.at[pl.ds(_NKEY, _N)])
        pltpu.sync_copy(ns_hbm.at[b], nsb)

        def _zero(j, c):
            buf[pl.ds(j * 16, 16)] = jnp.zeros((16,), _F32)
            return c
        lax.fori_loop(0, _N // 16, _zero, 0)

        bigi = jnp.int32(1 << 30)

        def _extract(ref, nvec, biggest):
            def body(j, c):
                bv, bi = c
                v = ref[pl.ds(j * 16, 16)]
                take = (v > bv) if biggest else (v < bv)
                gi = j * 16 + iota16
                return jnp.where(take, v, bv), jnp.where(take, gi, bi)
            init = jnp.full((16,), -3e38 if biggest else 3e38, _F32)
            bv, bi = lax.fori_loop(0, nvec, body,
                                   (init, jnp.zeros((16,), jnp.int32)))
            mval = bv[0]
            pick = jnp.maximum if biggest else jnp.minimum
            for i in range(1, 16):
                mval = pick(mval, bv[i])
            gi = bigi
            for i in range(16):
                gi = jnp.minimum(gi, jnp.where(bv[i] == mval, bi[i], bigi))
            return mval, gi

        def _put(ref, gi, val):
            g = (gi // 16) * 16
            lane = gi - g
            w = ref[pl.ds(g, 16)]
            ref[pl.ds(g, 16)] = jnp.where(iota16 == lane, val, w)

        a1, ai1 = _extract(nsb, _N // 16, True)
        _put(nsb, ai1, jnp.float32(-3e38))
        a2, _ = _extract(nsb, _N // 16, True)
        _put(nsb, ai1, a1)

        nv = []
        for _k in range(5):
            v, gi = _extract(nsb, _N // 16, False)
            nv.append(v)
            _put(nsb, gi, jnp.float32(3e38))

        gv0 = jnp.zeros((16,), jnp.int32)
        gv1 = jnp.zeros((16,), jnp.int32)
        gv2 = jnp.zeros((16,), jnp.int32)
        sv = jnp.zeros((16,), _F32)
        for _k in range(5):
            v, gi = _extract(buf, _NTOT // 16, True)
            _put(buf, gi, jnp.float32(-3e38))
            seg = gi // _N
            addr = b * _N + (gi - seg * _N)
            lk = iota16 == _k
            gv0 = jnp.where(lk, jnp.where(seg == 0, addr, gv0[_k]), gv0)
            gv1 = jnp.where(lk, jnp.where(seg == 1, addr, gv1[_k]), gv1)
            gv2 = jnp.where(lk, jnp.where(seg == 2, addr, gv2[_k]), gv2)
            sv = jnp.where(lk, seg.astype(_F32), sv)

        rn = jnp.zeros((16,), _F32)
        for _k in range(5):
            rn = jnp.where(iota16 == _k, nv[_k], rn)
        res[...] = rn
        pltpu.sync_copy(res, topn_out.at[b])

        ra = jnp.where(lane0, a1, jnp.where(iota16 == 1, a2, 0.0))
        res[...] = ra
        pltpu.sync_copy(res, topa_out.at[b])

        gidx[...] = jnp.where(gv0 == 0, b * _N, gv0)
        pltpu.async_copy(fcn_hbm.at[gidx], rows0, sem).wait()
        gidx[...] = jnp.where(gv1 == 0, b * _N, gv1)
        pltpu.async_copy(fca_hbm.at[gidx], rows1, sem).wait()
        gidx[...] = jnp.where(gv2 == 0, b * _N, gv2)
        pltpu.async_copy(fh_hbm.at[gidx], rows2, sem).wait()
        for _r in range(16):
            sr = sv[_r]
            for _c in range(_FD // 16):
                c0 = rows0[_r, pl.ds(_c * 16, 16)]
                c1 = rows1[_r, pl.ds(_c * 16, 16)]
                c2 = rows2[_r, pl.ds(_c * 16, 16)]
                rows0[_r, pl.ds(_c * 16, 16)] = jnp.where(
                    sr == 1.0, c1, jnp.where(sr == 2.0, c2, c0))
        pltpu.sync_copy(rows0, gath_out.at[b])
        for _r in range(16):
            srow = jnp.broadcast_to(sv[_r], (16,))
            for _c in range(_FD // 16):
                segx[_r, pl.ds(_c * 16, 16)] = srow
        pltpu.sync_copy(segx, segf_out.at[b])


@functools.lru_cache(maxsize=1)
def _kc_call():
    return pl.kernel(
        _kc_body,
        out_type=[
            jax.ShapeDtypeStruct((_B, 16), _F32),
            jax.ShapeDtypeStruct((_B, 16), _F32),
            jax.ShapeDtypeStruct((_B, 16, _FD), _F32),
            jax.ShapeDtypeStruct((_B, 16, _FD), _F32),
        ],
        mesh=plsc.VectorSubcoreMesh(core_axis_name="c", subcore_axis_name="s"),
        scratch_types=[
            pltpu.VMEM((_NTOT,), _F32),
            pltpu.VMEM((_N,), _F32),
            pltpu.VMEM((16,), jnp.int32),
            pltpu.VMEM((16, _FD), _F32),
            pltpu.VMEM((16, _FD), _F32),
            pltpu.VMEM((16, _FD), _F32),
            pltpu.VMEM((16, _FD), _F32),
            pltpu.VMEM((16,), _F32),
            pltpu.SemaphoreType.DMA,
        ],
    )


def _kd_attn(x, nreal):
    bi = lax.broadcasted_iota(jnp.int32, (_B * 16, _B * 16), 0)
    bj = lax.broadcasted_iota(jnp.int32, (_B * 16, _B * 16), 1)
    mask = ((bi // 16) == (bj // 16)) & ((bj % 16) < nreal)
    n = x * lax.rsqrt(jnp.sum(x * x, axis=1, keepdims=True))
    s = lax.dot_general(n, n, (((1,), (1,)), ((), ())),
                        preferred_element_type=_F32)
    e = jnp.where(mask, jnp.exp(s), 0.0)
    den = jnp.sum(e, axis=1, keepdims=True)
    o = lax.dot_general(e, x, (((1,), (0,)), ((), ())),
                        preferred_element_type=_F32)
    return o / den


def _kd_body(fcn_ref, g_ref, seg_ref, wq_ref, bq_ref, wk_ref, bk_ref,
             on_ref, oa_ref):
    dn = (((1,), (1,)), ((), ()))
    wq = wq_ref[...]
    bq = bq_ref[0]
    xnor = lax.dot_general(fcn_ref[...].reshape(_B * 16, _FD), wq, dn,
                           preferred_element_type=_F32) + bq
    g = g_ref[...].reshape(_B * 16, _FD)
    pq = lax.dot_general(g, wq, dn, preferred_element_type=_F32) + bq
    pk = lax.dot_general(g, wk_ref[...], dn,
                         preferred_element_type=_F32) + bk_ref[0]
    segc = seg_ref[...].reshape(_B * 16, _FD)
    xabn = jnp.where(segc >= 1.5, pk, pq)
    on_ref[...] = _kd_attn(xnor, 10).reshape(_B, 16, _FD)
    oa_ref[...] = _kd_attn(xabn, 5).reshape(_B, 16, _FD)


_kd_call = pl.pallas_call(
    _kd_body,
    grid=(1,),
    in_specs=[
        pl.BlockSpec((_B, 16, _FD), lambda s: (0, 0, 0)),
        pl.BlockSpec((_B, 16, _FD), lambda s: (0, 0, 0)),
        pl.BlockSpec((_B, 16, _FD), lambda s: (0, 0, 0)),
        pl.BlockSpec((_FD, _FD), lambda s: (0, 0)),
        pl.BlockSpec((1, _FD), lambda s: (0, 0)),
        pl.BlockSpec((_FD, _FD), lambda s: (0, 0)),
        pl.BlockSpec((1, _FD), lambda s: (0, 0)),
    ],
    out_specs=[
        pl.BlockSpec((_B, 16, _FD), lambda s: (0, 0, 0)),
        pl.BlockSpec((_B, 16, _FD), lambda s: (0, 0, 0)),
    ],
    out_shape=[
        jax.ShapeDtypeStruct((_B, 16, _FD), _F32),
        jax.ShapeDtypeStruct((_B, 16, _FD), _F32),
    ],
)


def kernel(nor_feat_conf_nor, feat_conf_nor, score_conf_nor, feat_conf_abn,
           score_conf_abn, feat_hard, score_hard, Wq, bq, Wk, bk):
    ftn = _kp_call(feat_conf_nor, feat_conf_abn, feat_hard,
                   Wq, bq.reshape(1, _FD), Wk, bk.reshape(1, _FD))
    ns = _ka_call(ftn, score_conf_abn.reshape(_B, 1, _N)).reshape(_B, _N)
    nnf = _kb_call(nor_feat_conf_nor)
    topn, topa, gath, segf = _kc_call()(
        ns, score_conf_abn, feat_conf_nor.reshape(_B * _N, _FD),
        feat_conf_abn.reshape(_B * _N, _FD), feat_hard.reshape(_B * _N, _FD))
    kdn, kda = _kd_call(feat_conf_nor, gath, segf, Wq, bq.reshape(1, _FD),
                        Wk, bk.reshape(1, _FD))
    return (ns, topn[:, :5], topa[:, :2], nnf, kdn[:, :10], kda[:, :5])


# speedup vs baseline: 1.5327x; 1.1292x over previous
import functools

import jax
import jax.numpy as jnp
from jax import lax
from jax.experimental import pallas as pl
from jax.experimental.pallas import tpu as pltpu
from jax.experimental.pallas import tpu_sc as plsc

_B, _N, _FD = 16, 1024, 128
_NKEY = 2 * _N
_NTOT = 3 * _N
_QB = 1024
_F32 = jnp.float32
_BF16 = jnp.bfloat16


def _kp_body(fcn_ref, fca_ref, fh_ref, wq_ref, bq_ref, wk_ref,
             bk_ref, ftn_out, ft_s):
    wq = wq_ref[...]
    bq = bq_ref[0]
    dn = (((1,), (1,)), ((), ()))
    ft_s[0:_N] = lax.dot_general(fcn_ref[0], wq, dn,
                                 preferred_element_type=_F32) + bq
    ft_s[_N:_NKEY] = lax.dot_general(fca_ref[0], wq, dn,
                                     preferred_element_type=_F32) + bq
    ft_s[_NKEY:_NTOT] = lax.dot_general(
        fh_ref[0], wk_ref[...], dn, preferred_element_type=_F32) + bk_ref[0]
    f = ft_s[...]
    ftn_out[0] = (f * lax.rsqrt(jnp.sum(f * f, axis=1, keepdims=True))
                  ).astype(_BF16)


_kp_call = pl.pallas_call(
    _kp_body,
    grid=(_B,),
    in_specs=[
        pl.BlockSpec((1, _N, _FD), lambda b: (b, 0, 0)),
        pl.BlockSpec((1, _N, _FD), lambda b: (b, 0, 0)),
        pl.BlockSpec((1, _N, _FD), lambda b: (b, 0, 0)),
        pl.BlockSpec((_FD, _FD), lambda b: (0, 0)),
        pl.BlockSpec((1, _FD), lambda b: (0, 0)),
        pl.BlockSpec((_FD, _FD), lambda b: (0, 0)),
        pl.BlockSpec((1, _FD), lambda b: (0, 0)),
    ],
    out_specs=pl.BlockSpec((1, _NTOT, _FD), lambda b: (b, 0, 0)),
    out_shape=jax.ShapeDtypeStruct((_B, _NTOT, _FD), _BF16),
    scratch_shapes=[pltpu.VMEM((_NTOT, _FD), _F32)],
)


def _ka_body(ftn_ref, sabn_ref, ns_out):
    qb = pl.program_id(1)
    kn = ftn_ref[0, 0:_NKEY, :]
    for h in range(2):
        base = _NKEY + qb * _QB + h * (_QB // 2)
        qn = ftn_ref[0, pl.ds(base, _QB // 2), :]
        s = lax.dot_general(qn, kn, (((1,), (1,)), ((), ())),
                            preferred_element_type=_F32)
        e = jnp.exp(s)
        den = jnp.sum(e, axis=1)
        num = lax.dot_general(e[:, _N:_NKEY], sabn_ref[0, 0],
                              (((1,), (0,)), ((), ())),
                              preferred_element_type=_F32)
        ns_out[0, 0, pl.ds(qb * _QB + h * (_QB // 2), _QB // 2)] = num / den


_ka_call = pl.pallas_call(
    _ka_body,
    grid=(_B, _N // _QB),
    in_specs=[
        pl.BlockSpec((1, _NTOT, _FD), lambda b, q: (b, 0, 0)),
        pl.BlockSpec((1, 1, _N), lambda b, q: (b, 0, 0)),
    ],
    out_specs=pl.BlockSpec((1, 1, _N), lambda b, q: (b, 0, 0)),
    out_shape=jax.ShapeDtypeStruct((_B, 1, _N), _F32),
)


def _kb_body(x_ref, o_ref, xn_s):
    qb = pl.program_id(1)

    @pl.when(qb == 0)
    def _norm():
        x = x_ref[0]
        xn_s[...] = (x * lax.rsqrt(jnp.sum(x * x, axis=1, keepdims=True))
                     ).astype(_BF16)

    qn = xn_s[pl.ds(qb * _QB, _QB), :]
    s = lax.dot_general(qn, xn_s[...], (((1,), (1,)), ((), ())),
                        preferred_element_type=_F32)
    e = jnp.exp(s)
    den = lax.dot_general(e, jnp.ones((_N,), _F32), (((1,), (0,)), ((), ())),
                          preferred_element_type=_F32)
    o = lax.dot_general(e, x_ref[0], (((1,), (0,)), ((), ())),
                        preferred_element_type=_F32)
    o_ref[0] = o / den[:, None]


_kb_call = pl.pallas_call(
    _kb_body,
    grid=(_B, _N // _QB),
    in_specs=[pl.BlockSpec((1, _N, _FD), lambda b, q: (b, 0, 0))],
    out_specs=pl.BlockSpec((1, _QB, _FD), lambda b, q: (b, q, 0)),
    out_shape=jax.ShapeDtypeStruct((_B, _N, _FD), _F32),
    scratch_shapes=[pltpu.VMEM((_N, _FD), _BF16)],
)


def _kc_body(ns_hbm, sabn_hbm, fcn_hbm, fca_hbm, fh_hbm,
             topn_out, topa_out, gath_out, segf_out,
             buf, nsb, gidx, rows0, rows1, rows2, segx, res, sem):
    wid = lax.axis_index("s") * 2 + lax.axis_index("c")

    @pl.when(wid < _B)
    def _work():
        b = wid
        iota16 = lax.iota(jnp.int32, 16)
        lane0 = iota16 == 0

        pltpu.sync_copy(sabn_hbm.at[b], buf.at[pl.ds(_N, _N)])
        pltpu.sync_copy(ns_hbm.at[b], buf.at[pl.ds(_NKEY, _N)])
        pltpu.sync_copy(ns_hbm.at[b], nsb)

        def _zero(j, c):
            buf[pl.ds(j * 16, 16)] = jnp.zeros((16,), _F32)
            return c
        lax.fori_loop(0, _N // 16, _zero, 0)

        bigi = jnp.int32(1 << 30)

        def _extract(ref, nvec, biggest):
            def body(j, c):
                bv, bi = c
                v = ref[pl.ds(j * 16, 16)]
                take = (v > bv) if biggest else (v < bv)
                gi = j * 16 + iota16
                return jnp.where(take, v, bv), jnp.where(take, gi, bi)
            init = jnp.full((16,), -3e38 if biggest else 3e38, _F32)
            bv, bi = lax.fori_loop(0, nvec, body,
                                   (init, jnp.zeros((16,), jnp.int32)))
            mval = bv[0]
            pick = jnp.maximum if biggest else jnp.minimum
            for i in range(1, 16):
                mval = pick(mval, bv[i])
            gi = bigi
            for i in range(16):
                gi = jnp.minimum(gi, jnp.where(bv[i] == mval, bi[i], bigi))
            return mval, gi

        def _put(ref, gi, val):
            g = (gi // 16) * 16
            lane = gi - g
            w = ref[pl.ds(g, 16)]
            ref[pl.ds(g, 16)] = jnp.where(iota16 == lane, val, w)

        a1, ai1 = _extract(nsb, _N // 16, True)
        _put(nsb, ai1, jnp.float32(-3e38))
        a2, _ = _extract(nsb, _N // 16, True)
        _put(nsb, ai1, a1)

        nv = []
        for _k in range(5):
            v, gi = _extract(nsb, _N // 16, False)
            nv.append(v)
            _put(nsb, gi, jnp.float32(3e38))

        gv0 = jnp.zeros((16,), jnp.int32)
        gv1 = jnp.zeros((16,), jnp.int32)
        gv2 = jnp.zeros((16,), jnp.int32)
        sv = jnp.zeros((16,), _F32)
        for _k in range(5):
            v, gi = _extract(buf, _NTOT // 16, True)
            _put(buf, gi, jnp.float32(-3e38))
            seg = gi // _N
            addr = b * _N + (gi - seg * _N)
            lk = iota16 == _k
            gv0 = jnp.where(lk, jnp.where(seg == 0, addr, gv0[_k]), gv0)
            gv1 = jnp.where(lk, jnp.where(seg == 1, addr, gv1[_k]), gv1)
            gv2 = jnp.where(lk, jnp.where(seg == 2, addr, gv2[_k]), gv2)
            sv = jnp.where(lk, seg.astype(_F32), sv)

        rn = jnp.zeros((16,), _F32)
        for _k in range(5):
            rn = jnp.where(iota16 == _k, nv[_k], rn)
        res[...] = rn
        pltpu.sync_copy(res, topn_out.at[b])

        ra = jnp.where(lane0, a1, jnp.where(iota16 == 1, a2, 0.0))
        res[...] = ra
        pltpu.sync_copy(res, topa_out.at[b])

        gidx[...] = jnp.where(gv0 == 0, b * _N, gv0)
        pltpu.async_copy(fcn_hbm.at[gidx], rows0, sem).wait()
        gidx[...] = jnp.where(gv1 == 0, b * _N, gv1)
        pltpu.async_copy(fca_hbm.at[gidx], rows1, sem).wait()
        gidx[...] = jnp.where(gv2 == 0, b * _N, gv2)
        pltpu.async_copy(fh_hbm.at[gidx], rows2, sem).wait()
        for _r in range(16):
            sr = sv[_r]
            for _c in range(_FD // 16):
                c0 = rows0[_r, pl.ds(_c * 16, 16)]
                c1 = rows1[_r, pl.ds(_c * 16, 16)]
                c2 = rows2[_r, pl.ds(_c * 16, 16)]
                rows0[_r, pl.ds(_c * 16, 16)] = jnp.where(
                    sr == 1.0, c1, jnp.where(sr == 2.0, c2, c0))
        pltpu.sync_copy(rows0, gath_out.at[b])
        for _r in range(16):
            srow = jnp.broadcast_to(sv[_r], (16,))
            for _c in range(_FD // 16):
                segx[_r, pl.ds(_c * 16, 16)] = srow
        pltpu.sync_copy(segx, segf_out.at[b])


@functools.lru_cache(maxsize=1)
def _kc_call():
    return pl.kernel(
        _kc_body,
        out_type=[
            jax.ShapeDtypeStruct((_B, 16), _F32),
            jax.ShapeDtypeStruct((_B, 16), _F32),
            jax.ShapeDtypeStruct((_B, 16, _FD), _F32),
            jax.ShapeDtypeStruct((_B, 16, _FD), _F32),
        ],
        mesh=plsc.VectorSubcoreMesh(core_axis_name="c", subcore_axis_name="s"),
        scratch_types=[
            pltpu.VMEM((_NTOT,), _F32),
            pltpu.VMEM((_N,), _F32),
            pltpu.VMEM((16,), jnp.int32),
            pltpu.VMEM((16, _FD), _F32),
            pltpu.VMEM((16, _FD), _F32),
            pltpu.VMEM((16, _FD), _F32),
            pltpu.VMEM((16, _FD), _F32),
            pltpu.VMEM((16,), _F32),
            pltpu.SemaphoreType.DMA,
        ],
    )


def _kd_attn(x, nreal):
    bi = lax.broadcasted_iota(jnp.int32, (_B * 16, _B * 16), 0)
    bj = lax.broadcasted_iota(jnp.int32, (_B * 16, _B * 16), 1)
    mask = ((bi // 16) == (bj // 16)) & ((bj % 16) < nreal)
    n = x * lax.rsqrt(jnp.sum(x * x, axis=1, keepdims=True))
    s = lax.dot_general(n, n, (((1,), (1,)), ((), ())),
                        preferred_element_type=_F32)
    e = jnp.where(mask, jnp.exp(s), 0.0)
    den = jnp.sum(e, axis=1, keepdims=True)
    o = lax.dot_general(e, x, (((1,), (0,)), ((), ())),
                        preferred_element_type=_F32)
    return o / den


def _kd_body(fcn_ref, g_ref, seg_ref, wq_ref, bq_ref, wk_ref, bk_ref,
             on_ref, oa_ref):
    dn = (((1,), (1,)), ((), ()))
    wq = wq_ref[...]
    bq = bq_ref[0]
    xnor = lax.dot_general(fcn_ref[...].reshape(_B * 16, _FD), wq, dn,
                           preferred_element_type=_F32) + bq
    g = g_ref[...].reshape(_B * 16, _FD)
    pq = lax.dot_general(g, wq, dn, preferred_element_type=_F32) + bq
    pk = lax.dot_general(g, wk_ref[...], dn,
                         preferred_element_type=_F32) + bk_ref[0]
    segc = seg_ref[...].reshape(_B * 16, _FD)
    xabn = jnp.where(segc >= 1.5, pk, pq)
    on_ref[...] = _kd_attn(xnor, 10).reshape(_B, 16, _FD)
    oa_ref[...] = _kd_attn(xabn, 5).reshape(_B, 16, _FD)


_kd_call = pl.pallas_call(
    _kd_body,
    grid=(1,),
    in_specs=[
        pl.BlockSpec((_B, 16, _FD), lambda s: (0, 0, 0)),
        pl.BlockSpec((_B, 16, _FD), lambda s: (0, 0, 0)),
        pl.BlockSpec((_B, 16, _FD), lambda s: (0, 0, 0)),
        pl.BlockSpec((_FD, _FD), lambda s: (0, 0)),
        pl.BlockSpec((1, _FD), lambda s: (0, 0)),
        pl.BlockSpec((_FD, _FD), lambda s: (0, 0)),
        pl.BlockSpec((1, _FD), lambda s: (0, 0)),
    ],
    out_specs=[
        pl.BlockSpec((_B, 16, _FD), lambda s: (0, 0, 0)),
        pl.BlockSpec((_B, 16, _FD), lambda s: (0, 0, 0)),
    ],
    out_shape=[
        jax.ShapeDtypeStruct((_B, 16, _FD), _F32),
        jax.ShapeDtypeStruct((_B, 16, _FD), _F32),
    ],
)


def kernel(nor_feat_conf_nor, feat_conf_nor, score_conf_nor, feat_conf_abn,
           score_conf_abn, feat_hard, score_hard, Wq, bq, Wk, bk):
    ftn = _kp_call(feat_conf_nor, feat_conf_abn, feat_hard,
                   Wq, bq.reshape(1, _FD), Wk, bk.reshape(1, _FD))
    ns = _ka_call(ftn, score_conf_abn.reshape(_B, 1, _N)).reshape(_B, _N)
    nnf = _kb_call(nor_feat_conf_nor)
    topn, topa, gath, segf = _kc_call()(
        ns, score_conf_abn, feat_conf_nor.reshape(_B * _N, _FD),
        feat_conf_abn.reshape(_B * _N, _FD), feat_hard.reshape(_B * _N, _FD))
    kdn, kda = _kd_call(feat_conf_nor, gath, segf, Wq, bq.reshape(1, _FD),
                        Wk, bk.reshape(1, _FD))
    return (ns, topn[:, :5], topa[:, :2], nnf, kdn[:, :10], kda[:, :5])


# speedup vs baseline: 1.7813x; 1.1622x over previous
import functools

import jax
import jax.numpy as jnp
from jax import lax
from jax.experimental import pallas as pl
from jax.experimental.pallas import tpu as pltpu
from jax.experimental.pallas import tpu_sc as plsc

_B, _N, _FD = 16, 1024, 128
_NKEY = 2 * _N
_NTOT = 3 * _N
_QB = 1024
_F32 = jnp.float32
_BF16 = jnp.bfloat16


def _ka_body(fcn_ref, fca_ref, fh_ref, sabn_ref, wq_ref, bq_ref, wk_ref,
             bk_ref, ns_out, ft_s, ftn_s):
    wq = wq_ref[...]
    bq = bq_ref[0]
    dn = (((1,), (1,)), ((), ()))
    ft_s[0:_N] = lax.dot_general(fcn_ref[0], wq, dn,
                                 preferred_element_type=_F32) + bq
    ft_s[_N:_NKEY] = lax.dot_general(fca_ref[0], wq, dn,
                                     preferred_element_type=_F32) + bq
    ft_s[_NKEY:_NTOT] = lax.dot_general(
        fh_ref[0], wk_ref[...], dn, preferred_element_type=_F32) + bk_ref[0]
    f = ft_s[...]
    ftn_s[...] = (f * lax.rsqrt(jnp.sum(f * f, axis=1, keepdims=True))
                  ).astype(_BF16)
    kn = ftn_s[0:_NKEY, :]
    sab = sabn_ref[0, 0]
    for h in range(2):
        qn = ftn_s[pl.ds(_NKEY + h * 512, 512), :]
        sm = lax.dot_general(qn, kn, (((1,), (1,)), ((), ())),
                             preferred_element_type=_F32)
        e = jnp.exp(sm)
        num = lax.dot_general(e[:, _N:_NKEY], sab, (((1,), (0,)), ((), ())),
                              preferred_element_type=_F32)
        den = lax.dot_general(e, jnp.ones((_NKEY,), _F32),
                              (((1,), (0,)), ((), ())),
                              preferred_element_type=_F32)
        ns_out[0, pl.ds(h * 4, 4), :] = (num / den).reshape(4, _FD)


_ka_call = pl.pallas_call(
    _ka_body,
    grid=(_B,),
    in_specs=[
        pl.BlockSpec((1, _N, _FD), lambda b: (b, 0, 0)),
        pl.BlockSpec((1, _N, _FD), lambda b: (b, 0, 0)),
        pl.BlockSpec((1, _N, _FD), lambda b: (b, 0, 0)),
        pl.BlockSpec((1, 1, _N), lambda b: (b, 0, 0)),
        pl.BlockSpec((_FD, _FD), lambda b: (0, 0)),
        pl.BlockSpec((1, _FD), lambda b: (0, 0)),
        pl.BlockSpec((_FD, _FD), lambda b: (0, 0)),
        pl.BlockSpec((1, _FD), lambda b: (0, 0)),
    ],
    out_specs=pl.BlockSpec((1, 8, _FD), lambda b: (b, 0, 0)),
    out_shape=jax.ShapeDtypeStruct((_B, 8, _FD), _F32),
    scratch_shapes=[
        pltpu.VMEM((_NTOT, _FD), _F32),
        pltpu.VMEM((_NTOT, _FD), _BF16),
    ],
)


def _kb_body(x_ref, o_ref, xn_s):
    qb = pl.program_id(1)

    @pl.when(qb == 0)
    def _norm():
        x = x_ref[0]
        xn_s[...] = (x * lax.rsqrt(jnp.sum(x * x, axis=1, keepdims=True))
                     ).astype(_BF16)

    qn = xn_s[pl.ds(qb * _QB, _QB), :]
    s = lax.dot_general(qn, xn_s[...], (((1,), (1,)), ((), ())),
                        preferred_element_type=_F32)
    e = jnp.exp(s)
    den = lax.dot_general(e, jnp.ones((_N,), _F32), (((1,), (0,)), ((), ())),
                          preferred_element_type=_F32)
    o = lax.dot_general(e, x_ref[0], (((1,), (0,)), ((), ())),
                        preferred_element_type=_F32)
    o_ref[0] = o / den[:, None]


_kb_call = pl.pallas_call(
    _kb_body,
    grid=(_B, _N // _QB),
    in_specs=[pl.BlockSpec((1, _N, _FD), lambda b, q: (b, 0, 0))],
    out_specs=pl.BlockSpec((1, _QB, _FD), lambda b, q: (b, q, 0)),
    out_shape=jax.ShapeDtypeStruct((_B, _N, _FD), _F32),
    scratch_shapes=[pltpu.VMEM((_N, _FD), _BF16)],
)


def _kc_body(ns_hbm, sabn_hbm, fcn_hbm, fca_hbm, fh_hbm,
             topn_out, topa_out, gath_out, segf_out,
             buf, nsb, gidx, rows0, rows1, rows2, segx, res, sem):
    wid = lax.axis_index("s") * 2 + lax.axis_index("c")

    @pl.when(wid < _B)
    def _work():
        b = wid
        iota16 = lax.iota(jnp.int32, 16)
        lane0 = iota16 == 0

        pltpu.sync_copy(sabn_hbm.at[b], buf.at[pl.ds(_N, _N)])
        pltpu.sync_copy(ns_hbm.at[b], buf.at[pl.ds(_NKEY, _N)])
        pltpu.sync_copy(ns_hbm.at[b], nsb)

        def _zero(j, c):
            buf[pl.ds(j * 16, 16)] = jnp.zeros((16,), _F32)
            return c
        lax.fori_loop(0, _N // 16, _zero, 0)

        bigi = jnp.int32(1 << 30)

        def _extract(ref, nvec, biggest):
            def body(j, c):
                bv, bi = c
                v = ref[pl.ds(j * 16, 16)]
                take = (v > bv) if biggest else (v < bv)
                gi = j * 16 + iota16
                return jnp.where(take, v, bv), jnp.where(take, gi, bi)
            init = jnp.full((16,), -3e38 if biggest else 3e38, _F32)
            bv, bi = lax.fori_loop(0, nvec, body,
                                   (init, jnp.zeros((16,), jnp.int32)))
            mval = bv[0]
            pick = jnp.maximum if biggest else jnp.minimum
            for i in range(1, 16):
                mval = pick(mval, bv[i])
            gi = bigi
            for i in range(16):
                gi = jnp.minimum(gi, jnp.where(bv[i] == mval, bi[i], bigi))
            return mval, gi

        def _put(ref, gi, val):
            g = (gi // 16) * 16
            lane = gi - g
            w = ref[pl.ds(g, 16)]
            ref[pl.ds(g, 16)] = jnp.where(iota16 == lane, val, w)

        a1, ai1 = _extract(nsb, _N // 16, True)
        _put(nsb, ai1, jnp.float32(-3e38))
        a2, _ = _extract(nsb, _N // 16, True)
        _put(nsb, ai1, a1)

        nv = []
        for _k in range(5):
            v, gi = _extract(nsb, _N // 16, False)
            nv.append(v)
            _put(nsb, gi, jnp.float32(3e38))

        gv0 = jnp.zeros((16,), jnp.int32)
        gv1 = jnp.zeros((16,), jnp.int32)
        gv2 = jnp.zeros((16,), jnp.int32)
        sv = jnp.zeros((16,), _F32)
        for _k in range(5):
            v, gi = _extract(buf, _NTOT // 16, True)
            _put(buf, gi, jnp.float32(-3e38))
            seg = gi // _N
            addr = b * _N + (gi - seg * _N)
            lk = iota16 == _k
            gv0 = jnp.where(lk, jnp.where(seg == 0, addr, gv0[_k]), gv0)
            gv1 = jnp.where(lk, jnp.where(seg == 1, addr, gv1[_k]), gv1)
            gv2 = jnp.where(lk, jnp.where(seg == 2, addr, gv2[_k]), gv2)
            sv = jnp.where(lk, seg.astype(_F32), sv)

        rn = jnp.zeros((16,), _F32)
        for _k in range(5):
            rn = jnp.where(iota16 == _k, nv[_k], rn)
        res[...] = rn
        pltpu.sync_copy(res, topn_out.at[b])

        ra = jnp.where(lane0, a1, jnp.where(iota16 == 1, a2, 0.0))
        res[...] = ra
        pltpu.sync_copy(res, topa_out.at[b])

        gidx[...] = jnp.where(gv0 == 0, b * _N, gv0)
        pltpu.async_copy(fcn_hbm.at[gidx], rows0, sem).wait()
        gidx[...] = jnp.where(gv1 == 0, b * _N, gv1)
        pltpu.async_copy(fca_hbm.at[gidx], rows1, sem).wait()
        gidx[...] = jnp.where(gv2 == 0, b * _N, gv2)
        pltpu.async_copy(fh_hbm.at[gidx], rows2, sem).wait()
        for _r in range(16):
            sr = sv[_r]
            for _c in range(_FD // 16):
                c0 = rows0[_r, pl.ds(_c * 16, 16)]
                c1 = rows1[_r, pl.ds(_c * 16, 16)]
                c2 = rows2[_r, pl.ds(_c * 16, 16)]
                rows0[_r, pl.ds(_c * 16, 16)] = jnp.where(
                    sr == 1.0, c1, jnp.where(sr == 2.0, c2, c0))
        pltpu.sync_copy(rows0, gath_out.at[b])
        for _r in range(16):
            srow = jnp.broadcast_to(sv[_r], (16,))
            for _c in range(_FD // 16):
                segx[_r, pl.ds(_c * 16, 16)] = srow
        pltpu.sync_copy(segx, segf_out.at[b])


@functools.lru_cache(maxsize=1)
def _kc_call():
    return pl.kernel(
        _kc_body,
        out_type=[
            jax.ShapeDtypeStruct((_B, 16), _F32),
            jax.ShapeDtypeStruct((_B, 16), _F32),
            jax.ShapeDtypeStruct((_B, 16, _FD), _F32),
            jax.ShapeDtypeStruct((_B, 16, _FD), _F32),
        ],
        mesh=plsc.VectorSubcoreMesh(core_axis_name="c", subcore_axis_name="s"),
        scratch_types=[
            pltpu.VMEM((_NTOT,), _F32),
            pltpu.VMEM((_N,), _F32),
            pltpu.VMEM((16,), jnp.int32),
            pltpu.VMEM((16, _FD), _F32),
            pltpu.VMEM((16, _FD), _F32),
            pltpu.VMEM((16, _FD), _F32),
            pltpu.VMEM((16, _FD), _F32),
            pltpu.VMEM((16,), _F32),
            pltpu.SemaphoreType.DMA,
        ],
    )


def _kd_attn(x, nreal):
    bi = lax.broadcasted_iota(jnp.int32, (_B * 16, _B * 16), 0)
    bj = lax.broadcasted_iota(jnp.int32, (_B * 16, _B * 16), 1)
    mask = ((bi // 16) == (bj // 16)) & ((bj % 16) < nreal)
    n = x * lax.rsqrt(jnp.sum(x * x, axis=1, keepdims=True))
    s = lax.dot_general(n, n, (((1,), (1,)), ((), ())),
                        preferred_element_type=_F32)
    e = jnp.where(mask, jnp.exp(s), 0.0)
    den = jnp.sum(e, axis=1, keepdims=True)
    o = lax.dot_general(e, x, (((1,), (0,)), ((), ())),
                        preferred_element_type=_F32)
    return o / den


def _kd_body(fcn_ref, g_ref, seg_ref, wq_ref, bq_ref, wk_ref, bk_ref,
             on_ref, oa_ref):
    dn = (((1,), (1,)), ((), ()))
    wq = wq_ref[...]
    bq = bq_ref[0]
    xnor = lax.dot_general(fcn_ref[...].reshape(_B * 16, _FD), wq, dn,
                           preferred_element_type=_F32) + bq
    g = g_ref[...].reshape(_B * 16, _FD)
    pq = lax.dot_general(g, wq, dn, preferred_element_type=_F32) + bq
    pk = lax.dot_general(g, wk_ref[...], dn,
                         preferred_element_type=_F32) + bk_ref[0]
    segc = seg_ref[...].reshape(_B * 16, _FD)
    xabn = jnp.where(segc >= 1.5, pk, pq)
    on_ref[...] = _kd_attn(xnor, 10).reshape(_B, 16, _FD)
    oa_ref[...] = _kd_attn(xabn, 5).reshape(_B, 16, _FD)


_kd_call = pl.pallas_call(
    _kd_body,
    grid=(1,),
    in_specs=[
        pl.BlockSpec((_B, 16, _FD), lambda s: (0, 0, 0)),
        pl.BlockSpec((_B, 16, _FD), lambda s: (0, 0, 0)),
        pl.BlockSpec((_B, 16, _FD), lambda s: (0, 0, 0)),
        pl.BlockSpec((_FD, _FD), lambda s: (0, 0)),
        pl.BlockSpec((1, _FD), lambda s: (0, 0)),
        pl.BlockSpec((_FD, _FD), lambda s: (0, 0)),
        pl.BlockSpec((1, _FD), lambda s: (0, 0)),
    ],
    out_specs=[
        pl.BlockSpec((_B, 16, _FD), lambda s: (0, 0, 0)),
        pl.BlockSpec((_B, 16, _FD), lambda s: (0, 0, 0)),
    ],
    out_shape=[
        jax.ShapeDtypeStruct((_B, 16, _FD), _F32),
        jax.ShapeDtypeStruct((_B, 16, _FD), _F32),
    ],
)


def kernel(nor_feat_conf_nor, feat_conf_nor, score_conf_nor, feat_conf_abn,
           score_conf_abn, feat_hard, score_hard, Wq, bq, Wk, bk):
    ns = _ka_call(feat_conf_nor, feat_conf_abn, feat_hard,
                  score_conf_abn.reshape(_B, 1, _N), Wq, bq.reshape(1, _FD),
                  Wk, bk.reshape(1, _FD)).reshape(_B, _N)
    nnf = _kb_call(nor_feat_conf_nor)
    topn, topa, gath, segf = _kc_call()(
        ns, score_conf_abn, feat_conf_nor.reshape(_B * _N, _FD),
        feat_conf_abn.reshape(_B * _N, _FD), feat_hard.reshape(_B * _N, _FD))
    kdn, kda = _kd_call(feat_conf_nor, gath, segf, Wq, bq.reshape(1, _FD),
                        Wk, bk.reshape(1, _FD))
    return (ns, topn[:, :5], topa[:, :2], nnf, kdn[:, :10], kda[:, :5])


# speedup vs baseline: 1.7855x; 1.0024x over previous
import functools

import jax
import jax.numpy as jnp
from jax import lax
from jax.experimental import pallas as pl
from jax.experimental.pallas import tpu as pltpu
from jax.experimental.pallas import tpu_sc as plsc

_B, _N, _FD = 16, 1024, 128
_NKEY = 2 * _N
_NTOT = 3 * _N
_QB = 1024
_F32 = jnp.float32
_BF16 = jnp.bfloat16


def _ka_body(fcn_ref, fca_ref, fh_ref, sabn_ref, wq_ref, bq_ref, wk_ref,
             bk_ref, ns_out, ftn_s):
    wq = wq_ref[...]
    bq = bq_ref[0]
    dn = (((1,), (1,)), ((), ()))
    f = jnp.concatenate([
        lax.dot_general(fcn_ref[0], wq, dn, preferred_element_type=_F32) + bq,
        lax.dot_general(fca_ref[0], wq, dn, preferred_element_type=_F32) + bq,
        lax.dot_general(fh_ref[0], wk_ref[...], dn,
                        preferred_element_type=_F32) + bk_ref[0],
    ], axis=0)
    ftn_s[...] = (f * lax.rsqrt(jnp.sum(f * f, axis=1, keepdims=True))
                  ).astype(_BF16)
    kn = ftn_s[0:_NKEY, :]
    sab = sabn_ref[0, 0]
    for h in range(2):
        qn = ftn_s[pl.ds(_NKEY + h * 512, 512), :]
        sm = lax.dot_general(qn, kn, (((1,), (1,)), ((), ())),
                             preferred_element_type=_F32)
        e = jnp.exp(sm)
        num = lax.dot_general(e[:, _N:_NKEY], sab, (((1,), (0,)), ((), ())),
                              preferred_element_type=_F32)
        den = lax.dot_general(e, jnp.ones((_NKEY,), _F32),
                              (((1,), (0,)), ((), ())),
                              preferred_element_type=_F32)
        ns_out[0, pl.ds(h * 4, 4), :] = (num / den).reshape(4, _FD)


_ka_call = pl.pallas_call(
    _ka_body,
    grid=(_B,),
    in_specs=[
        pl.BlockSpec((1, _N, _FD), lambda b: (b, 0, 0)),
        pl.BlockSpec((1, _N, _FD), lambda b: (b, 0, 0)),
        pl.BlockSpec((1, _N, _FD), lambda b: (b, 0, 0)),
        pl.BlockSpec((1, 1, _N), lambda b: (b, 0, 0)),
        pl.BlockSpec((_FD, _FD), lambda b: (0, 0)),
        pl.BlockSpec((1, _FD), lambda b: (0, 0)),
        pl.BlockSpec((_FD, _FD), lambda b: (0, 0)),
        pl.BlockSpec((1, _FD), lambda b: (0, 0)),
    ],
    out_specs=pl.BlockSpec((1, 8, _FD), lambda b: (b, 0, 0)),
    out_shape=jax.ShapeDtypeStruct((_B, 8, _FD), _F32),
    scratch_shapes=[
        pltpu.VMEM((_NTOT, _FD), _BF16),
    ],
)


def _kb_body(x_ref, o_ref, xn_s):
    qb = pl.program_id(1)

    @pl.when(qb == 0)
    def _norm():
        x = x_ref[0]
        xn_s[...] = (x * lax.rsqrt(jnp.sum(x * x, axis=1, keepdims=True))
                     ).astype(_BF16)

    qn = xn_s[pl.ds(qb * _QB, _QB), :]
    s = lax.dot_general(qn, xn_s[...], (((1,), (1,)), ((), ())),
                        preferred_element_type=_F32)
    e = jnp.exp(s)
    den = lax.dot_general(e, jnp.ones((_N,), _F32), (((1,), (0,)), ((), ())),
                          preferred_element_type=_F32)
    o = lax.dot_general(e, x_ref[0], (((1,), (0,)), ((), ())),
                        preferred_element_type=_F32)
    o_ref[0] = o / den[:, None]


_kb_call = pl.pallas_call(
    _kb_body,
    grid=(_B, _N // _QB),
    in_specs=[pl.BlockSpec((1, _N, _FD), lambda b, q: (b, 0, 0))],
    out_specs=pl.BlockSpec((1, _QB, _FD), lambda b, q: (b, q, 0)),
    out_shape=jax.ShapeDtypeStruct((_B, _N, _FD), _F32),
    scratch_shapes=[pltpu.VMEM((_N, _FD), _BF16)],
)


def _kc_body(ns_hbm, sabn_hbm, fcn_hbm, fca_hbm, fh_hbm,
             topn_out, topa_out, gath_out, segf_out,
             buf, nsb, gidx, rows0, rows1, rows2, segx, res, sem):
    wid = lax.axis_index("s") * 2 + lax.axis_index("c")

    @pl.when(wid < _B)
    def _work():
        b = wid
        iota16 = lax.iota(jnp.int32, 16)
        lane0 = iota16 == 0

        pltpu.sync_copy(sabn_hbm.at[b], buf.at[pl.ds(_N, _N)])
        pltpu.sync_copy(ns_hbm.at[b], buf.at[pl.ds(_NKEY, _N)])
        pltpu.sync_copy(ns_hbm.at[b], nsb)

        def _zero(j, c):
            buf[pl.ds(j * 16, 16)] = jnp.zeros((16,), _F32)
            return c
        lax.fori_loop(0, _N // 16, _zero, 0)

        bigi = jnp.int32(1 << 30)

        def _extract(ref, nvec, biggest):
            def body(j, c):
                bv, bi = c
                v = ref[pl.ds(j * 16, 16)]
                take = (v > bv) if biggest else (v < bv)
                gi = j * 16 + iota16
                return jnp.where(take, v, bv), jnp.where(take, gi, bi)
            init = jnp.full((16,), -3e38 if biggest else 3e38, _F32)
            bv, bi = lax.fori_loop(0, nvec, body,
                                   (init, jnp.zeros((16,), jnp.int32)))
            mval = bv[0]
            pick = jnp.maximum if biggest else jnp.minimum
            for i in range(1, 16):
                mval = pick(mval, bv[i])
            gi = bigi
            for i in range(16):
                gi = jnp.minimum(gi, jnp.where(bv[i] == mval, bi[i], bigi))
            return mval, gi

        def _put(ref, gi, val):
            g = (gi // 16) * 16
            lane = gi - g
            w = ref[pl.ds(g, 16)]
            ref[pl.ds(g, 16)] = jnp.where(iota16 == lane, val, w)

        a1, ai1 = _extract(nsb, _N // 16, True)
        _put(nsb, ai1, jnp.float32(-3e38))
        a2, _ = _extract(nsb, _N // 16, True)
        _put(nsb, ai1, a1)

        nv = []
        for _k in range(5):
            v, gi = _extract(nsb, _N // 16, False)
            nv.append(v)
            _put(nsb, gi, jnp.float32(3e38))

        gv0 = jnp.zeros((16,), jnp.int32)
        gv1 = jnp.zeros((16,), jnp.int32)
        gv2 = jnp.zeros((16,), jnp.int32)
        sv = jnp.zeros((16,), _F32)
        for _k in range(5):
            v, gi = _extract(buf, _NTOT // 16, True)
            _put(buf, gi, jnp.float32(-3e38))
            seg = gi // _N
            addr = b * _N + (gi - seg * _N)
            lk = iota16 == _k
            gv0 = jnp.where(lk, jnp.where(seg == 0, addr, gv0[_k]), gv0)
            gv1 = jnp.where(lk, jnp.where(seg == 1, addr, gv1[_k]), gv1)
            gv2 = jnp.where(lk, jnp.where(seg == 2, addr, gv2[_k]), gv2)
            sv = jnp.where(lk, seg.astype(_F32), sv)

        rn = jnp.zeros((16,), _F32)
        for _k in range(5):
            rn = jnp.where(iota16 == _k, nv[_k], rn)
        res[...] = rn
        pltpu.sync_copy(res, topn_out.at[b])

        ra = jnp.where(lane0, a1, jnp.where(iota16 == 1, a2, 0.0))
        res[...] = ra
        pltpu.sync_copy(res, topa_out.at[b])

        gidx[...] = jnp.where(gv0 == 0, b * _N, gv0)
        pltpu.async_copy(fcn_hbm.at[gidx], rows0, sem).wait()
        gidx[...] = jnp.where(gv1 == 0, b * _N, gv1)
        pltpu.async_copy(fca_hbm.at[gidx], rows1, sem).wait()
        gidx[...] = jnp.where(gv2 == 0, b * _N, gv2)
        pltpu.async_copy(fh_hbm.at[gidx], rows2, sem).wait()
        for _r in range(16):
            sr = sv[_r]
            for _c in range(_FD // 16):
                c0 = rows0[_r, pl.ds(_c * 16, 16)]
                c1 = rows1[_r, pl.ds(_c * 16, 16)]
                c2 = rows2[_r, pl.ds(_c * 16, 16)]
                rows0[_r, pl.ds(_c * 16, 16)] = jnp.where(
                    sr == 1.0, c1, jnp.where(sr == 2.0, c2, c0))
        pltpu.sync_copy(rows0, gath_out.at[b])
        for _r in range(16):
            srow = jnp.broadcast_to(sv[_r], (16,))
            for _c in range(_FD // 16):
                segx[_r, pl.ds(_c * 16, 16)] = srow
        pltpu.sync_copy(segx, segf_out.at[b])


@functools.lru_cache(maxsize=1)
def _kc_call():
    return pl.kernel(
        _kc_body,
        out_type=[
            jax.ShapeDtypeStruct((_B, 16), _F32),
            jax.ShapeDtypeStruct((_B, 16), _F32),
            jax.ShapeDtypeStruct((_B, 16, _FD), _F32),
            jax.ShapeDtypeStruct((_B, 16, _FD), _F32),
        ],
        mesh=plsc.VectorSubcoreMesh(core_axis_name="c", subcore_axis_name="s"),
        scratch_types=[
            pltpu.VMEM((_NTOT,), _F32),
            pltpu.VMEM((_N,), _F32),
            pltpu.VMEM((16,), jnp.int32),
            pltpu.VMEM((16, _FD), _F32),
            pltpu.VMEM((16, _FD), _F32),
            pltpu.VMEM((16, _FD), _F32),
            pltpu.VMEM((16, _FD), _F32),
            pltpu.VMEM((16,), _F32),
            pltpu.SemaphoreType.DMA,
        ],
    )


def _kd_attn(x, nreal):
    bi = lax.broadcasted_iota(jnp.int32, (_B * 16, _B * 16), 0)
    bj = lax.broadcasted_iota(jnp.int32, (_B * 16, _B * 16), 1)
    mask = ((bi // 16) == (bj // 16)) & ((bj % 16) < nreal)
    n = x * lax.rsqrt(jnp.sum(x * x, axis=1, keepdims=True))
    s = lax.dot_general(n, n, (((1,), (1,)), ((), ())),
                        preferred_element_type=_F32)
    e = jnp.where(mask, jnp.exp(s), 0.0)
    den = jnp.sum(e, axis=1, keepdims=True)
    o = lax.dot_general(e, x, (((1,), (0,)), ((), ())),
                        preferred_element_type=_F32)
    return o / den


def _kd_body(fcn_ref, g_ref, seg_ref, wq_ref, bq_ref, wk_ref, bk_ref,
             on_ref, oa_ref):
    dn = (((1,), (1,)), ((), ()))
    wq = wq_ref[...]
    bq = bq_ref[0]
    xnor = lax.dot_general(fcn_ref[...].reshape(_B * 16, _FD), wq, dn,
                           preferred_element_type=_F32) + bq
    g = g_ref[...].reshape(_B * 16, _FD)
    pq = lax.dot_general(g, wq, dn, preferred_element_type=_F32) + bq
    pk = lax.dot_general(g, wk_ref[...], dn,
                         preferred_element_type=_F32) + bk_ref[0]
    segc = seg_ref[...].reshape(_B * 16, _FD)
    xabn = jnp.where(segc >= 1.5, pk, pq)
    on_ref[...] = _kd_attn(xnor, 10).reshape(_B, 16, _FD)
    oa_ref[...] = _kd_attn(xabn, 5).reshape(_B, 16, _FD)


_kd_call = pl.pallas_call(
    _kd_body,
    grid=(1,),
    in_specs=[
        pl.BlockSpec((_B, 16, _FD), lambda s: (0, 0, 0)),
        pl.BlockSpec((_B, 16, _FD), lambda s: (0, 0, 0)),
        pl.BlockSpec((_B, 16, _FD), lambda s: (0, 0, 0)),
        pl.BlockSpec((_FD, _FD), lambda s: (0, 0)),
        pl.BlockSpec((1, _FD), lambda s: (0, 0)),
        pl.BlockSpec((_FD, _FD), lambda s: (0, 0)),
        pl.BlockSpec((1, _FD), lambda s: (0, 0)),
    ],
    out_specs=[
        pl.BlockSpec((_B, 16, _FD), lambda s: (0, 0, 0)),
        pl.BlockSpec((_B, 16, _FD), lambda s: (0, 0, 0)),
    ],
    out_shape=[
        jax.ShapeDtypeStruct((_B, 16, _FD), _F32),
        jax.ShapeDtypeStruct((_B, 16, _FD), _F32),
    ],
)


def kernel(nor_feat_conf_nor, feat_conf_nor, score_conf_nor, feat_conf_abn,
           score_conf_abn, feat_hard, score_hard, Wq, bq, Wk, bk):
    ns = _ka_call(feat_conf_nor, feat_conf_abn, feat_hard,
                  score_conf_abn.reshape(_B, 1, _N), Wq, bq.reshape(1, _FD),
                  Wk, bk.reshape(1, _FD)).reshape(_B, _N)
    nnf = _kb_call(nor_feat_conf_nor)
    topn, topa, gath, segf = _kc_call()(
        ns, score_conf_abn, feat_conf_nor.reshape(_B * _N, _FD),
        feat_conf_abn.reshape(_B * _N, _FD), feat_hard.reshape(_B * _N, _FD))
    kdn, kda = _kd_call(feat_conf_nor, gath, segf, Wq, bq.reshape(1, _FD),
                        Wk, bk.reshape(1, _FD))
    return (ns, topn[:, :5], topa[:, :2], nnf, kdn[:, :10], kda[:, :5])
